# baseline (speedup 1.0000x reference)
import functools

import jax
import jax.numpy as jnp
from jax import lax
from jax.experimental import pallas as pl
from jax.experimental.pallas import tpu as pltpu
from jax.experimental.pallas import tpu_sc as plsc

B = 32
SIZE = 32
P = SIZE * SIZE
D = 2048
WN = D // 8
LEVELS = 256
GR = 16
NG = P // GR
LANES = 16
NJ = WN // LANES
PW = 512


def _sc_body(x_hbm, valp_hbm, posp_hbm, out_hbm,
             x_v, idx_v, posp_v, acc_v, out_v,
             buf0, buf1, buf2, buf3, sem0, sem1, sem2, sem3):
    wid = lax.axis_index("s") * 2 + lax.axis_index("c")
    pltpu.sync_copy(x_hbm.at[wid], x_v)
    pltpu.sync_copy(posp_hbm, posp_v)

    zero = jnp.zeros((LANES,), jnp.int32)

    def qbody(i, _):
        off = i * LANES
        f = x_v[pl.ds(off, LANES)] * float(LEVELS - 1) + 0.5
        t = f.astype(jnp.int32)
        tie = (t.astype(jnp.float32) == f) & ((t & 1) == 1)
        t = t - jnp.where(tie, 1, 0)
        idx_v[pl.ds(off, LANES)] = jnp.clip(t, 0, LEVELS - 1)
        return 0
    lax.fori_loop(0, P // LANES, qbody, 0)

    def zbody(j, _):
        off = j * LANES
        for k in range(8):
            acc_v[k, pl.ds(off, LANES)] = zero
        return 0
    lax.fori_loop(0, NJ, zbody, 0)

    def gather(g, buf, sem):
        return pltpu.async_copy(
            valp_hbm.at[idx_v.at[pl.ds(g * GR, GR)]], buf, sem)

    def wait_gather(buf, sem):
        pltpu.make_async_copy(
            valp_hbm.at[idx_v.at[pl.ds(0, GR)]], buf, sem).wait()

    gather(0, buf0, sem0)
    gather(1, buf1, sem1)
    gather(2, buf2, sem2)
    gather(3, buf3, sem3)

    def accumulate(g, buf):
        def jbody(j, _):
            off = j * LANES
            for seg0 in (0, 8):
                partial = zero
                for r in range(seg0, seg0 + 8):
                    rem = (8 - (r % 8)) % 8
                    w0 = (rem * PW + WN - 2 * g - (r + rem) // 8 + off)
                    partial = partial + (buf[r, pl.ds(off, LANES)]
                                         ^ posp_v[pl.ds(w0, LANES)])
                for k in range(8):
                    nib = (partial >> (4 * k)) & 15 if k else partial & 15
                    acc_v[k, pl.ds(off, LANES)] = (
                        acc_v[k, pl.ds(off, LANES)] + nib)
            return 0
        lax.fori_loop(0, NJ, jbody, 0, unroll=2)

    NQ = NG // 4

    def outer(q, _):
        g0 = q * 4
        wait_gather(buf0, sem0)
        accumulate(g0, buf0)

        @pl.when(q < NQ - 1)
        def _():
            gather(g0 + 4, buf0, sem0)

        wait_gather(buf1, sem1)
        accumulate(g0 + 1, buf1)

        @pl.when(q < NQ - 1)
        def _():
            gather(g0 + 5, buf1, sem1)

        wait_gather(buf2, sem2)
        accumulate(g0 + 2, buf2)

        @pl.when(q < NQ - 1)
        def _():
            gather(g0 + 6, buf2, sem2)

        wait_gather(buf3, sem3)
        accumulate(g0 + 3, buf3)

        @pl.when(q < NQ - 1)
        def _():
            gather(g0 + 7, buf3, sem3)
        return 0

    lax.fori_loop(0, NQ, outer, 0)

    half_p = P // 2

    def tbody(j, _):
        off = j * LANES
        word = zero
        for k in range(8):
            bit = jnp.where(acc_v[k, pl.ds(off, LANES)] > half_p, 1, 0)
            word = word | (bit << (4 * k)) if k else bit
        out_v[pl.ds(off, LANES)] = word
        return 0
    lax.fori_loop(0, NJ, tbody, 0)
    pltpu.sync_copy(out_v, out_hbm.at[wid])


_SC_MESH = plsc.VectorSubcoreMesh(core_axis_name="c", subcore_axis_name="s")

_sc_call = functools.partial(
    pl.kernel,
    mesh=_SC_MESH,
    out_type=jax.ShapeDtypeStruct((B, WN), jnp.int32),
    scratch_types=[
        pltpu.VMEM((P,), jnp.float32),
        pltpu.VMEM((P,), jnp.int32),
        pltpu.VMEM((8 * PW,), jnp.int32),
        pltpu.VMEM((8, WN), jnp.int32),
        pltpu.VMEM((WN,), jnp.int32),
        pltpu.VMEM((GR, WN), jnp.int32),
        pltpu.VMEM((GR, WN), jnp.int32),
        pltpu.VMEM((GR, WN), jnp.int32),
        pltpu.VMEM((GR, WN), jnp.int32),
        pltpu.SemaphoreType.DMA,
        pltpu.SemaphoreType.DMA,
        pltpu.SemaphoreType.DMA,
        pltpu.SemaphoreType.DMA,
    ],
)(_sc_body)


@jax.jit
def kernel(x, position_weight, value_weight):
    shifts = (jnp.arange(8, dtype=jnp.int32) * 4)[None, :]
    valp = jnp.sum(
        value_weight.reshape(LEVELS * WN, 8) << shifts, axis=1,
        dtype=jnp.int32).reshape(LEVELS, WN)
    brow = position_weight[0]
    b6 = jnp.concatenate([brow, brow, brow])
    posp = jnp.stack([
        jnp.sum(lax.slice(b6, (r,), (r + 8 * PW,)).reshape(PW, 8) << shifts,
                axis=1, dtype=jnp.int32)
        for r in range(8)
    ]).reshape(8 * PW)

    out_nib = _sc_call(x.reshape(B, P), valp, posp)
    return ((out_nib[:, :, None] >> shifts.reshape(1, 1, 8)) & 1).reshape(B, D)


# speedup vs baseline: 1.2356x; 1.2356x over previous
import functools

import jax
import jax.numpy as jnp
from jax import lax
from jax.experimental import pallas as pl
from jax.experimental.pallas import tpu as pltpu
from jax.experimental.pallas import tpu_sc as plsc

B = 32
SIZE = 32
P = SIZE * SIZE
D = 2048
WN = D // 8
LEVELS = 256
GR = 16
NG = P // GR
LANES = 16
NJ = WN // LANES
PW = 512


def _sc_body(x_hbm, valp_hbm, posp_hbm, out_hbm,
             x_v, idx_v, posp_v, acc_v, out_v,
             buf0, buf1, buf2, buf3, sem0, sem1, sem2, sem3):
    wid = lax.axis_index("s") * 2 + lax.axis_index("c")
    pltpu.sync_copy(x_hbm.at[wid], x_v)
    pltpu.sync_copy(posp_hbm, posp_v)

    zero = jnp.zeros((LANES,), jnp.int32)

    def qbody(i, _):
        off = i * LANES
        f = x_v[pl.ds(off, LANES)] * float(LEVELS - 1) + 0.5
        t = f.astype(jnp.int32)
        tie = (t.astype(jnp.float32) == f) & ((t & 1) == 1)
        t = t - jnp.where(tie, 1, 0)
        idx_v[pl.ds(off, LANES)] = jnp.clip(t, 0, LEVELS - 1)
        return 0
    lax.fori_loop(0, P // LANES, qbody, 0)

    def zbody(j, _):
        off = j * LANES
        for k in range(8):
            acc_v[k, pl.ds(off, LANES)] = zero
        return 0
    lax.fori_loop(0, NJ, zbody, 0)

    def gather(g, buf, sem):
        return pltpu.async_copy(
            valp_hbm.at[idx_v.at[pl.ds(g * GR, GR)]], buf, sem)

    def wait_gather(buf, sem):
        pltpu.make_async_copy(
            valp_hbm.at[idx_v.at[pl.ds(0, GR)]], buf, sem).wait()

    gather(0, buf0, sem0)
    gather(1, buf1, sem1)
    gather(2, buf2, sem2)
    gather(3, buf3, sem3)

    def accumulate(g, buf):
        def jbody(j, _):
            off = j * LANES
            parts = []
            for seg0 in (0, 8):
                partial = zero
                for r in range(seg0, seg0 + 8):
                    rem = (8 - (r % 8)) % 8
                    w0 = (rem * PW + WN - 2 * g - (r + rem) // 8 + off)
                    partial = partial + (buf[r, pl.ds(off, LANES)]
                                         ^ posp_v[pl.ds(w0, LANES)])
                parts.append(partial)
            pa, pb = parts
            for k in range(8):
                sh = 4 * k
                nib = (((pa >> sh) & 15) + ((pb >> sh) & 15)
                       if k else (pa & 15) + (pb & 15))
                acc_v[k, pl.ds(off, LANES)] = (
                    acc_v[k, pl.ds(off, LANES)] + nib)
            return 0
        lax.fori_loop(0, NJ, jbody, 0, unroll=2)

    NQ = NG // 4

    def outer(q, _):
        g0 = q * 4
        wait_gather(buf0, sem0)
        accumulate(g0, buf0)

        @pl.when(q < NQ - 1)
        def _():
            gather(g0 + 4, buf0, sem0)

        wait_gather(buf1, sem1)
        accumulate(g0 + 1, buf1)

        @pl.when(q < NQ - 1)
        def _():
            gather(g0 + 5, buf1, sem1)

        wait_gather(buf2, sem2)
        accumulate(g0 + 2, buf2)

        @pl.when(q < NQ - 1)
        def _():
            gather(g0 + 6, buf2, sem2)

        wait_gather(buf3, sem3)
        accumulate(g0 + 3, buf3)

        @pl.when(q < NQ - 1)
        def _():
            gather(g0 + 7, buf3, sem3)
        return 0

    lax.fori_loop(0, NQ, outer, 0)

    half_p = P // 2

    def tbody(j, _):
        off = j * LANES
        word = zero
        for k in range(8):
            bit = jnp.where(acc_v[k, pl.ds(off, LANES)] > half_p, 1, 0)
            word = word | (bit << (4 * k)) if k else bit
        out_v[pl.ds(off, LANES)] = word
        return 0
    lax.fori_loop(0, NJ, tbody, 0)
    pltpu.sync_copy(out_v, out_hbm.at[wid])


_SC_MESH = plsc.VectorSubcoreMesh(core_axis_name="c", subcore_axis_name="s")

_sc_call = functools.partial(
    pl.kernel,
    mesh=_SC_MESH,
    out_type=jax.ShapeDtypeStruct((B, WN), jnp.int32),
    scratch_types=[
        pltpu.VMEM((P,), jnp.float32),
        pltpu.VMEM((P,), jnp.int32),
        pltpu.VMEM((8 * PW,), jnp.int32),
        pltpu.VMEM((8, WN), jnp.int32),
        pltpu.VMEM((WN,), jnp.int32),
        pltpu.VMEM((GR, WN), jnp.int32),
        pltpu.VMEM((GR, WN), jnp.int32),
        pltpu.VMEM((GR, WN), jnp.int32),
        pltpu.VMEM((GR, WN), jnp.int32),
        pltpu.SemaphoreType.DMA,
        pltpu.SemaphoreType.DMA,
        pltpu.SemaphoreType.DMA,
        pltpu.SemaphoreType.DMA,
    ],
)(_sc_body)


@jax.jit
def kernel(x, position_weight, value_weight):
    shifts = (jnp.arange(8, dtype=jnp.int32) * 4)[None, :]
    valp = jnp.sum(
        value_weight.reshape(LEVELS * WN, 8) << shifts, axis=1,
        dtype=jnp.int32).reshape(LEVELS, WN)
    brow = position_weight[0]
    b6 = jnp.concatenate([brow, brow, brow])
    posp = jnp.stack([
        jnp.sum(lax.slice(b6, (r,), (r + 8 * PW,)).reshape(PW, 8) << shifts,
                axis=1, dtype=jnp.int32)
        for r in range(8)
    ]).reshape(8 * PW)

    out_nib = _sc_call(x.reshape(B, P), valp, posp)
    return ((out_nib[:, :, None] >> shifts.reshape(1, 1, 8)) & 1).reshape(B, D)


# speedup vs baseline: 1.2497x; 1.0115x over previous
import functools

import jax
import jax.numpy as jnp
from jax import lax
from jax.experimental import pallas as pl
from jax.experimental.pallas import tpu as pltpu
from jax.experimental.pallas import tpu_sc as plsc

B = 32
SIZE = 32
P = SIZE * SIZE
D = 2048
WN = D // 8
LEVELS = 256
GR = 16
NG = P // GR
LANES = 16
NJ = WN // LANES
PW = 512


def _sc_body(x_hbm, valp_hbm, posp_hbm, out_hbm,
             x_v, idx_v, posp_v, acc_v, out_v,
             buf0, buf1, buf2, buf3, sem0, sem1, sem2, sem3):
    wid = lax.axis_index("s") * 2 + lax.axis_index("c")
    pltpu.sync_copy(x_hbm.at[wid], x_v)
    pltpu.sync_copy(posp_hbm, posp_v)

    zero = jnp.zeros((LANES,), jnp.int32)

    def qbody(i, _):
        off = i * LANES
        f = x_v[pl.ds(off, LANES)] * float(LEVELS - 1) + 0.5
        t = f.astype(jnp.int32)
        tie = (t.astype(jnp.float32) == f) & ((t & 1) == 1)
        t = t - jnp.where(tie, 1, 0)
        idx_v[pl.ds(off, LANES)] = jnp.clip(t, 0, LEVELS - 1)
        return 0
    lax.fori_loop(0, P // LANES, qbody, 0)

    def zbody(j, _):
        off = j * LANES
        for k in range(8):
            acc_v[k, pl.ds(off, LANES)] = zero
        return 0
    lax.fori_loop(0, NJ, zbody, 0)

    def gather(g, buf, sem):
        return pltpu.async_copy(
            valp_hbm.at[idx_v.at[pl.ds(g * GR, GR)]], buf, sem)

    def wait_gather(buf, sem):
        pltpu.make_async_copy(
            valp_hbm.at[idx_v.at[pl.ds(0, GR)]], buf, sem).wait()

    gather(0, buf0, sem0)
    gather(1, buf1, sem1)
    gather(2, buf2, sem2)
    gather(3, buf3, sem3)

    def accumulate(g, buf):
        def jbody(j, _):
            off = j * LANES
            parts = []
            for seg0 in (0, 8):
                partial = zero
                for r in range(seg0, seg0 + 8):
                    rem = (8 - (r % 8)) % 8
                    w0 = (rem * PW + WN - 2 * g - (r + rem) // 8 + off)
                    partial = partial + (buf[r, pl.ds(off, LANES)]
                                         ^ posp_v[pl.ds(w0, LANES)])
                parts.append(partial)
            pa, pb = parts
            for k in range(8):
                sh = 4 * k
                nib = (((pa >> sh) & 15) + ((pb >> sh) & 15)
                       if k else (pa & 15) + (pb & 15))
                acc_v[k, pl.ds(off, LANES)] = (
                    acc_v[k, pl.ds(off, LANES)] + nib)
            return 0
        lax.fori_loop(0, NJ, jbody, 0, unroll=4)

    NQ = NG // 4

    def outer(q, _):
        g0 = q * 4
        wait_gather(buf0, sem0)
        accumulate(g0, buf0)

        @pl.when(q < NQ - 1)
        def _():
            gather(g0 + 4, buf0, sem0)

        wait_gather(buf1, sem1)
        accumulate(g0 + 1, buf1)

        @pl.when(q < NQ - 1)
        def _():
            gather(g0 + 5, buf1, sem1)

        wait_gather(buf2, sem2)
        accumulate(g0 + 2, buf2)

        @pl.when(q < NQ - 1)
        def _():
            gather(g0 + 6, buf2, sem2)

        wait_gather(buf3, sem3)
        accumulate(g0 + 3, buf3)

        @pl.when(q < NQ - 1)
        def _():
            gather(g0 + 7, buf3, sem3)
        return 0

    lax.fori_loop(0, NQ, outer, 0)

    half_p = P // 2

    def tbody(j, _):
        off = j * LANES
        word = zero
        for k in range(8):
            bit = jnp.where(acc_v[k, pl.ds(off, LANES)] > half_p, 1, 0)
            word = word | (bit << (4 * k)) if k else bit
        out_v[pl.ds(off, LANES)] = word
        return 0
    lax.fori_loop(0, NJ, tbody, 0)
    pltpu.sync_copy(out_v, out_hbm.at[wid])


_SC_MESH = plsc.VectorSubcoreMesh(core_axis_name="c", subcore_axis_name="s")

_sc_call = functools.partial(
    pl.kernel,
    mesh=_SC_MESH,
    out_type=jax.ShapeDtypeStruct((B, WN), jnp.int32),
    scratch_types=[
        pltpu.VMEM((P,), jnp.float32),
        pltpu.VMEM((P,), jnp.int32),
        pltpu.VMEM((8 * PW,), jnp.int32),
        pltpu.VMEM((8, WN), jnp.int32),
        pltpu.VMEM((WN,), jnp.int32),
        pltpu.VMEM((GR, WN), jnp.int32),
        pltpu.VMEM((GR, WN), jnp.int32),
        pltpu.VMEM((GR, WN), jnp.int32),
        pltpu.VMEM((GR, WN), jnp.int32),
        pltpu.SemaphoreType.DMA,
        pltpu.SemaphoreType.DMA,
        pltpu.SemaphoreType.DMA,
        pltpu.SemaphoreType.DMA,
    ],
)(_sc_body)


@jax.jit
def kernel(x, position_weight, value_weight):
    shifts = (jnp.arange(8, dtype=jnp.int32) * 4)[None, :]
    valp = jnp.sum(
        value_weight.reshape(LEVELS * WN, 8) << shifts, axis=1,
        dtype=jnp.int32).reshape(LEVELS, WN)
    brow = position_weight[0]
    b6 = jnp.concatenate([brow, brow, brow])
    posp = jnp.stack([
        jnp.sum(lax.slice(b6, (r,), (r + 8 * PW,)).reshape(PW, 8) << shifts,
                axis=1, dtype=jnp.int32)
        for r in range(8)
    ]).reshape(8 * PW)

    out_nib = _sc_call(x.reshape(B, P), valp, posp)
    return ((out_nib[:, :, None] >> shifts.reshape(1, 1, 8)) & 1).reshape(B, D)


# speedup vs baseline: 1.8316x; 1.4656x over previous
import functools

import jax
import jax.numpy as jnp
from jax import lax
from jax.experimental import pallas as pl
from jax.experimental.pallas import tpu as pltpu
from jax.experimental.pallas import tpu_sc as plsc

B = 32
SIZE = 32
P = SIZE * SIZE
D = 2048
W = D // 4
LEVELS = 256
GR = 16
NG = P // GR
LANES = 16
NJ = W // LANES


def _sc_body(x_hbm, valp_hbm, posp_hbm, out_hbm,
             x_v, idx_v, posp_v, acc_v, out_v,
             buf0, buf1, buf2, buf3, sem0, sem1, sem2, sem3):
    wid = lax.axis_index("s") * 2 + lax.axis_index("c")
    pltpu.sync_copy(x_hbm.at[wid], x_v)
    pltpu.sync_copy(posp_hbm, posp_v)

    zero = jnp.zeros((LANES,), jnp.int32)

    def qbody(i, _):
        off = i * LANES
        f = x_v[pl.ds(off, LANES)] * float(LEVELS - 1) + 0.5
        t = f.astype(jnp.int32)
        tie = (t.astype(jnp.float32) == f) & ((t & 1) == 1)
        t = t - jnp.where(tie, 1, 0)
        idx_v[pl.ds(off, LANES)] = jnp.clip(t, 0, LEVELS - 1)
        return 0
    lax.fori_loop(0, P // LANES, qbody, 0)

    def zbody(j, _):
        off = j * LANES
        for k in range(4):
            acc_v[k, pl.ds(off, LANES)] = zero
        return 0
    lax.fori_loop(0, NJ, zbody, 0)

    def gather(g, buf, sem):
        return pltpu.async_copy(
            valp_hbm.at[idx_v.at[pl.ds(g * GR, GR)]], buf, sem)

    def wait_gather(buf, sem):
        pltpu.make_async_copy(
            valp_hbm.at[idx_v.at[pl.ds(0, GR)]], buf, sem).wait()

    gather(0, buf0, sem0)
    gather(1, buf1, sem1)
    gather(2, buf2, sem2)
    gather(3, buf3, sem3)

    def accumulate2(g0, bufa, bufb):
        def jbody(j, _):
            off = j * LANES
            partial = zero
            for half, buf in ((0, bufa), (1, bufb)):
                for r in range(GR):
                    rem = (4 - (r % 4)) % 4
                    w0 = (rem * 2 * W + W - 4 * (g0 + half)
                          - (r + rem) // 4 + off)
                    partial = partial + (buf[r, pl.ds(off, LANES)]
                                         ^ posp_v[pl.ds(w0, LANES)])
            for k in range(4):
                byte = (partial >> (8 * k)) & 255 if k else partial & 255
                acc_v[k, pl.ds(off, LANES)] = acc_v[k, pl.ds(off, LANES)] + byte
            return 0
        lax.fori_loop(0, NJ, jbody, 0, unroll=2)

    NQ = NG // 4

    def outer(q, _):
        g0 = q * 4
        wait_gather(buf0, sem0)
        wait_gather(buf1, sem1)
        accumulate2(g0, buf0, buf1)

        @pl.when(q < NQ - 1)
        def _():
            gather(g0 + 4, buf0, sem0)
            gather(g0 + 5, buf1, sem1)

        wait_gather(buf2, sem2)
        wait_gather(buf3, sem3)
        accumulate2(g0 + 2, buf2, buf3)

        @pl.when(q < NQ - 1)
        def _():
            gather(g0 + 6, buf2, sem2)
            gather(g0 + 7, buf3, sem3)
        return 0

    lax.fori_loop(0, NQ, outer, 0)

    half_p = P // 2

    def tbody(j, _):
        off = j * LANES
        word = zero
        for k in range(4):
            bit = jnp.where(acc_v[k, pl.ds(off, LANES)] > half_p, 1, 0)
            word = word | (bit << (8 * k)) if k else bit
        out_v[pl.ds(off, LANES)] = word
        return 0
    lax.fori_loop(0, NJ, tbody, 0)
    pltpu.sync_copy(out_v, out_hbm.at[wid])


_SC_MESH = plsc.VectorSubcoreMesh(core_axis_name="c", subcore_axis_name="s")

_sc_call = functools.partial(
    pl.kernel,
    mesh=_SC_MESH,
    out_type=jax.ShapeDtypeStruct((B, W), jnp.int32),
    scratch_types=[
        pltpu.VMEM((P,), jnp.float32),
        pltpu.VMEM((P,), jnp.int32),
        pltpu.VMEM((4 * 2 * W,), jnp.int32),
        pltpu.VMEM((4, W), jnp.int32),
        pltpu.VMEM((W,), jnp.int32),
        pltpu.VMEM((GR, W), jnp.int32),
        pltpu.VMEM((GR, W), jnp.int32),
        pltpu.VMEM((GR, W), jnp.int32),
        pltpu.VMEM((GR, W), jnp.int32),
        pltpu.SemaphoreType.DMA,
        pltpu.SemaphoreType.DMA,
        pltpu.SemaphoreType.DMA,
        pltpu.SemaphoreType.DMA,
    ],
)(_sc_body)


@jax.jit
def kernel(x, position_weight, value_weight):
    val8 = value_weight.astype(jnp.int8)
    valp = lax.bitcast_convert_type(val8.reshape(LEVELS, W, 4), jnp.int32)
    brow = position_weight[0:1, :]
    b6 = jnp.concatenate([brow, brow, brow], axis=1)
    pos4 = jnp.concatenate(
        [lax.slice(b6, (0, r), (1, r + 2 * D)) for r in range(4)], axis=0
    ).astype(jnp.int8)
    posp = lax.bitcast_convert_type(
        pos4.reshape(4, 2 * W, 4), jnp.int32).reshape(4 * 2 * W)

    out_words = _sc_call(x.reshape(B, P), valp, posp)
    out8 = lax.bitcast_convert_type(out_words, jnp.int8)
    return out8.reshape(B, D).astype(jnp.int32)
